# baseline (device time: 108737 ns/iter reference)
import jax
import jax.numpy as jnp
from jax import lax
from jax.experimental import pallas as pl
from jax.experimental.pallas import tpu as pltpu

N_DEV = 32
CAP = 48
N_CHUNK = 4
CH = N_DEV // N_CHUNK


def kernel(x, router_W, route_idx, expert_W):
    n_tok, d = x.shape
    e_per, _, h = expert_W.shape
    n_exp = N_DEV * e_per

    assert e_per == 2

    def body(x_ref, rw_ref, idx_ref, ew_ref, out_ref,
             send_ref, recv_ref, res_ref, ret_ref,
             p1_send, p1_recv, p2_send, p2_recv):
        my = lax.axis_index("i")

        barrier_sem = pltpu.get_barrier_semaphore()
        for off in range(1, N_DEV):
            peer = lax.rem(my + off, N_DEV)
            pl.semaphore_signal(
                barrier_sem, inc=1,
                device_id=(peer,), device_id_type=pl.DeviceIdType.MESH,
            )
        pl.semaphore_wait(barrier_sem, N_DEV - 1)

        xv32 = x_ref[...]
        xv = xv32.astype(jnp.bfloat16)
        scores = jnp.dot(xv32, rw_ref[...], preferred_element_type=jnp.float32)
        mx = jnp.max(scores, axis=-1, keepdims=True)
        p = jnp.exp(scores - mx)
        probs = p / jnp.sum(p, axis=-1, keepdims=True)
        e0c = idx_ref[:, 0:1]
        e1c = idx_ref[:, 1:2]
        eid = lax.broadcasted_iota(jnp.int32, (n_tok, n_exp), 1)
        one0 = eid == e0c
        one1 = eid == e1c
        mask_all = one0 | one1
        g0 = jnp.sum(probs * one0.astype(jnp.float32), axis=-1, keepdims=True)
        g1 = jnp.sum(probs * one1.astype(jnp.float32), axis=-1, keepdims=True)
        gs = g0 + g1
        w0 = g0 / gs
        w1 = g1 / gs

        ti = lax.broadcasted_iota(jnp.int32, (n_tok, n_tok), 0)
        tj = lax.broadcasted_iota(jnp.int32, (n_tok, n_tok), 1)
        low_tri = (ti > tj).astype(jnp.bfloat16)
        ranks = jnp.dot(low_tri, mask_all.astype(jnp.bfloat16),
                        preferred_element_type=jnp.float32).astype(jnp.int32)
        e0r = jnp.transpose(e0c)
        e1r = jnp.transpose(e1c)
        eidT = lax.broadcasted_iota(jnp.int32, (n_exp, n_tok), 0)
        mT = (eidT == e0r) | (eidT == e1r)
        up_tri = (ti < tj).astype(jnp.bfloat16)
        ranksT = jnp.dot(mT.astype(jnp.bfloat16), up_tri,
                         preferred_element_type=jnp.float32).astype(jnp.int32)

        c_row = lax.broadcasted_iota(jnp.int32, (CAP, n_tok), 0)
        s_parts = []
        for j in range(N_DEV):
            for k in range(e_per):
                e = lax.rem((my + j) * e_per, n_exp) + k
                r_sel = jnp.sum(jnp.where(eidT == e, ranksT, 0),
                                axis=0, keepdims=True)
                m_sel = (e0r == e) | (e1r == e)
                s_parts.append(
                    jnp.where((c_row == r_sel) & m_sel, 1.0, 0.0
                              ).astype(jnp.bfloat16))
        s_all = jnp.concatenate(s_parts, axis=0)
        send_ref[...] = jnp.dot(
            s_all, xv, preferred_element_type=jnp.float32
        ).astype(jnp.bfloat16).reshape(N_DEV, e_per, CAP, d)

        p1 = {}
        for j in range(1, N_DEV):
            dst = lax.rem(my + j, N_DEV)
            dsc = pltpu.make_async_remote_copy(
                src_ref=send_ref.at[pl.ds(j, 1)],
                dst_ref=recv_ref.at[pl.ds(j, 1)],
                send_sem=p1_send.at[j - 1],
                recv_sem=p1_recv.at[j - 1],
                device_id=(dst,),
                device_id_type=pl.DeviceIdType.MESH,
            )
            dsc.start()
            p1[j] = dsc
        recv_ref[pl.ds(0, 1)] = send_ref[pl.ds(0, 1)]

        c_col = lax.broadcasted_iota(jnp.int32, (n_tok, CAP), 1)
        g_chunks = []
        for mc in range(N_CHUNK):
            g_parts = []
            for m in range(mc * CH, (mc + 1) * CH):
                for k in range(e_per):
                    e = lax.rem((my - m + N_DEV) * e_per, n_exp) + k
                    rk_sel = jnp.sum(jnp.where(eid == e, ranks, 0),
                                     axis=1, keepdims=True)
                    msk_sel = (e0c == e) | (e1c == e)
                    ce = (jnp.where(e0c == e, w0, 0.0)
                          + jnp.where(e1c == e, w1, 0.0))
                    g_parts.append(
                        jnp.where((c_col == rk_sel) & msk_sel, ce, 0.0
                                  ).astype(jnp.bfloat16))
            g_chunks.append(jnp.concatenate(g_parts, axis=1))

        wb = ew_ref[...].astype(jnp.bfloat16)
        p2 = {}
        for c in range(N_CHUNK):
            lo = c * CH
            for j in range(max(lo, 1), lo + CH):
                p1[j].wait_recv()
            rv = recv_ref[pl.ds(lo, CH)]
            outs = []
            for k in range(e_per):
                xin = rv[:, k].reshape(CH * CAP, d)
                ok = jnp.dot(xin, wb[k], preferred_element_type=jnp.float32)
                outs.append(ok.astype(jnp.bfloat16).reshape(CH, CAP, h))
            res_ref[pl.ds(lo, CH)] = jnp.stack(outs, axis=1)
            for j in range(max(lo, 1), lo + CH):
                dst = lax.rem(my - j + N_DEV, N_DEV)
                dsc = pltpu.make_async_remote_copy(
                    src_ref=res_ref.at[pl.ds(j, 1)],
                    dst_ref=ret_ref.at[pl.ds(N_DEV - j, 1)],
                    send_sem=p2_send.at[j - 1],
                    recv_sem=p2_recv.at[N_DEV - j - 1],
                    device_id=(dst,),
                    device_id_type=pl.DeviceIdType.MESH,
                )
                dsc.start()
                p2[N_DEV - j] = dsc
        ret_ref[pl.ds(0, 1)] = res_ref[pl.ds(0, 1)]

        acc = jnp.zeros((n_tok, h), jnp.float32)
        for mc in range(N_CHUNK - 1, -1, -1):
            lo = mc * CH
            for m in range(max(lo, 1), lo + CH):
                p2[m].wait_recv()
            rets = ret_ref[pl.ds(lo, CH)].reshape(CH * e_per * CAP, h)
            acc = acc + jnp.dot(g_chunks[mc], rets,
                                preferred_element_type=jnp.float32)
        out_ref[...] = acc

        for j in range(1, N_DEV):
            p1[j].wait_send()
            p2[N_DEV - j].wait_send()

    return pl.pallas_call(
        body,
        out_shape=jax.ShapeDtypeStruct((n_tok, h), jnp.float32),
        in_specs=[
            pl.BlockSpec(memory_space=pltpu.VMEM),
            pl.BlockSpec(memory_space=pltpu.VMEM),
            pl.BlockSpec(memory_space=pltpu.VMEM),
            pl.BlockSpec(memory_space=pltpu.VMEM),
        ],
        out_specs=pl.BlockSpec(memory_space=pltpu.VMEM),
        scratch_shapes=[
            pltpu.VMEM((N_DEV, e_per, CAP, d), jnp.bfloat16),
            pltpu.VMEM((N_DEV, e_per, CAP, d), jnp.bfloat16),
            pltpu.VMEM((N_DEV, e_per, CAP, h), jnp.bfloat16),
            pltpu.VMEM((N_DEV, e_per, CAP, h), jnp.bfloat16),
            pltpu.SemaphoreType.DMA((N_DEV - 1,)),
            pltpu.SemaphoreType.DMA((N_DEV - 1,)),
            pltpu.SemaphoreType.DMA((N_DEV - 1,)),
            pltpu.SemaphoreType.DMA((N_DEV - 1,)),
        ],
        compiler_params=pltpu.CompilerParams(
            collective_id=0,
            vmem_limit_bytes=100 * 1024 * 1024,
        ),
    )(x, router_W, route_idx, expert_W)


# device time: 100369 ns/iter; 1.0834x vs baseline; 1.0834x over previous
import jax
import jax.numpy as jnp
from jax import lax
from jax.experimental import pallas as pl
from jax.experimental.pallas import tpu as pltpu

N_DEV = 32
CAP = 48
N_CHUNK = 4
CH = N_DEV // N_CHUNK


def kernel(x, router_W, route_idx, expert_W):
    n_tok, d = x.shape
    e_per, _, h = expert_W.shape
    n_exp = N_DEV * e_per

    assert e_per == 2

    def body(x_ref, rw_ref, idx_ref, ew_ref, out_ref,
             send_ref, recv_ref, res_ref, ret_ref,
             p1_send, p1_recv, p2_send, p2_recv):
        my = lax.axis_index("i")

        barrier_sem = pltpu.get_barrier_semaphore()
        for off in range(1, N_DEV):
            peer = lax.rem(my + off, N_DEV)
            pl.semaphore_signal(
                barrier_sem, inc=1,
                device_id=(peer,), device_id_type=pl.DeviceIdType.MESH,
            )
        pl.semaphore_wait(barrier_sem, N_DEV - 1)

        xv32 = x_ref[...]
        xv = xv32.astype(jnp.bfloat16)
        scores = jnp.dot(xv32, rw_ref[...], preferred_element_type=jnp.float32)
        mx = jnp.max(scores, axis=-1, keepdims=True)
        p = jnp.exp(scores - mx)
        probs = p / jnp.sum(p, axis=-1, keepdims=True)
        e0c = idx_ref[:, 0:1]
        e1c = idx_ref[:, 1:2]
        eid = lax.broadcasted_iota(jnp.int32, (n_tok, n_exp), 1)
        one0 = eid == e0c
        one1 = eid == e1c
        mask_all = one0 | one1
        g0 = jnp.sum(probs * one0.astype(jnp.float32), axis=-1, keepdims=True)
        g1 = jnp.sum(probs * one1.astype(jnp.float32), axis=-1, keepdims=True)
        gs = g0 + g1
        w0 = g0 / gs
        w1 = g1 / gs

        ti = lax.broadcasted_iota(jnp.int32, (n_tok, n_tok), 0)
        tj = lax.broadcasted_iota(jnp.int32, (n_tok, n_tok), 1)
        low_tri = (ti > tj).astype(jnp.bfloat16)
        ranks = jnp.dot(low_tri, mask_all.astype(jnp.bfloat16),
                        preferred_element_type=jnp.float32).astype(jnp.int32)
        e0r = jnp.transpose(e0c)
        e1r = jnp.transpose(e1c)
        eidT = lax.broadcasted_iota(jnp.int32, (n_exp, n_tok), 0)
        mT = (eidT == e0r) | (eidT == e1r)
        up_tri = (ti < tj).astype(jnp.bfloat16)
        ranksT = jnp.dot(mT.astype(jnp.bfloat16), up_tri,
                         preferred_element_type=jnp.float32).astype(jnp.int32)

        c_row = lax.broadcasted_iota(jnp.int32, (CAP, n_tok), 0)
        s_parts = []
        for e in range(n_exp):
            s_parts.append(
                jnp.where((c_row == ranksT[e:e + 1, :]) & mT[e:e + 1, :],
                          1.0, 0.0).astype(jnp.bfloat16))
        s_all = jnp.concatenate(s_parts, axis=0)
        send_vals = jnp.dot(
            s_all, xv, preferred_element_type=jnp.float32
        ).astype(jnp.bfloat16)
        send_ref[...] = pltpu.roll(
            send_vals, -my * e_per * CAP, axis=0
        ).reshape(N_DEV, e_per, CAP, d)

        p1 = {}
        for j in range(1, N_DEV):
            dst = lax.rem(my + j, N_DEV)
            dsc = pltpu.make_async_remote_copy(
                src_ref=send_ref.at[pl.ds(j, 1)],
                dst_ref=recv_ref.at[pl.ds(j, 1)],
                send_sem=p1_send.at[j - 1],
                recv_sem=p1_recv.at[j - 1],
                device_id=(dst,),
                device_id_type=pl.DeviceIdType.MESH,
            )
            dsc.start()
            p1[j] = dsc
        recv_ref[pl.ds(0, 1)] = send_ref[pl.ds(0, 1)]

        c_col = lax.broadcasted_iota(jnp.int32, (n_tok, CAP), 1)
        g_parts = []
        for t in range(N_DEV):
            dd = (-t) % N_DEV
            for k in range(e_per):
                e = dd * e_per + k
                ce = (jnp.where(e0c == e, w0, 0.0)
                      + jnp.where(e1c == e, w1, 0.0))
                g_parts.append(
                    jnp.where((c_col == ranks[:, e:e + 1])
                              & mask_all[:, e:e + 1], ce, 0.0
                              ).astype(jnp.bfloat16))
        g_desc = jnp.concatenate(g_parts, axis=1)
        g_m = pltpu.roll(g_desc, my * e_per * CAP, axis=1)
        g_chunks = [
            g_m[:, mc * CH * e_per * CAP:(mc + 1) * CH * e_per * CAP]
            for mc in range(N_CHUNK)
        ]

        wb = ew_ref[...].astype(jnp.bfloat16)
        p2 = {}
        for c in range(N_CHUNK):
            lo = c * CH
            for j in range(max(lo, 1), lo + CH):
                p1[j].wait_recv()
            rv = recv_ref[pl.ds(lo, CH)]
            outs = []
            for k in range(e_per):
                xin = rv[:, k].reshape(CH * CAP, d)
                ok = jnp.dot(xin, wb[k], preferred_element_type=jnp.float32)
                outs.append(ok.astype(jnp.bfloat16).reshape(CH, CAP, h))
            res_ref[pl.ds(lo, CH)] = jnp.stack(outs, axis=1)
            for j in range(max(lo, 1), lo + CH):
                dst = lax.rem(my - j + N_DEV, N_DEV)
                dsc = pltpu.make_async_remote_copy(
                    src_ref=res_ref.at[pl.ds(j, 1)],
                    dst_ref=ret_ref.at[pl.ds(N_DEV - j, 1)],
                    send_sem=p2_send.at[j - 1],
                    recv_sem=p2_recv.at[N_DEV - j - 1],
                    device_id=(dst,),
                    device_id_type=pl.DeviceIdType.MESH,
                )
                dsc.start()
                p2[N_DEV - j] = dsc
        ret_ref[pl.ds(0, 1)] = res_ref[pl.ds(0, 1)]

        acc = jnp.zeros((n_tok, h), jnp.float32)
        for mc in range(N_CHUNK - 1, -1, -1):
            lo = mc * CH
            for m in range(max(lo, 1), lo + CH):
                p2[m].wait_recv()
            rets = ret_ref[pl.ds(lo, CH)].reshape(CH * e_per * CAP, h)
            acc = acc + jnp.dot(g_chunks[mc], rets,
                                preferred_element_type=jnp.float32)
        out_ref[...] = acc

        for j in range(1, N_DEV):
            p1[j].wait_send()
            p2[N_DEV - j].wait_send()

    return pl.pallas_call(
        body,
        out_shape=jax.ShapeDtypeStruct((n_tok, h), jnp.float32),
        in_specs=[
            pl.BlockSpec(memory_space=pltpu.VMEM),
            pl.BlockSpec(memory_space=pltpu.VMEM),
            pl.BlockSpec(memory_space=pltpu.VMEM),
            pl.BlockSpec(memory_space=pltpu.VMEM),
        ],
        out_specs=pl.BlockSpec(memory_space=pltpu.VMEM),
        scratch_shapes=[
            pltpu.VMEM((N_DEV, e_per, CAP, d), jnp.bfloat16),
            pltpu.VMEM((N_DEV, e_per, CAP, d), jnp.bfloat16),
            pltpu.VMEM((N_DEV, e_per, CAP, h), jnp.bfloat16),
            pltpu.VMEM((N_DEV, e_per, CAP, h), jnp.bfloat16),
            pltpu.SemaphoreType.DMA((N_DEV - 1,)),
            pltpu.SemaphoreType.DMA((N_DEV - 1,)),
            pltpu.SemaphoreType.DMA((N_DEV - 1,)),
            pltpu.SemaphoreType.DMA((N_DEV - 1,)),
        ],
        compiler_params=pltpu.CompilerParams(
            collective_id=0,
            vmem_limit_bytes=100 * 1024 * 1024,
        ),
    )(x, router_W, route_idx, expert_W)


# device time: 100114 ns/iter; 1.0861x vs baseline; 1.0025x over previous
import jax
import jax.numpy as jnp
from jax import lax
from jax.experimental import pallas as pl
from jax.experimental.pallas import tpu as pltpu

N_DEV = 32
CAP = 48
N_CHUNK = 4
CH = N_DEV // N_CHUNK


def kernel(x, router_W, route_idx, expert_W):
    n_tok, d = x.shape
    e_per, _, h = expert_W.shape
    n_exp = N_DEV * e_per

    assert e_per == 2

    def body(x_ref, rw_ref, idx_ref, ew_ref, out_ref,
             send_ref, recv_ref, res_ref, ret_ref,
             p1_send, p1_recv, p2_send, p2_recv):
        my = lax.axis_index("i")

        barrier_sem = pltpu.get_barrier_semaphore()
        for off in range(1, N_DEV):
            peer = lax.rem(my + off, N_DEV)
            pl.semaphore_signal(
                barrier_sem, inc=1,
                device_id=(peer,), device_id_type=pl.DeviceIdType.MESH,
            )
        pl.semaphore_wait(barrier_sem, N_DEV - 1)

        xv32 = x_ref[...]
        xv = xv32.astype(jnp.bfloat16)
        scores = jnp.dot(xv32, rw_ref[...], preferred_element_type=jnp.float32)
        mx = jnp.max(scores, axis=-1, keepdims=True)
        p = jnp.exp(scores - mx)
        probs = p / jnp.sum(p, axis=-1, keepdims=True)
        e0c = idx_ref[:, 0:1]
        e1c = idx_ref[:, 1:2]
        eid = lax.broadcasted_iota(jnp.int32, (n_tok, n_exp), 1)
        one0 = eid == e0c
        one1 = eid == e1c
        mask_all = one0 | one1
        g0 = jnp.sum(probs * one0.astype(jnp.float32), axis=-1, keepdims=True)
        g1 = jnp.sum(probs * one1.astype(jnp.float32), axis=-1, keepdims=True)
        gs = g0 + g1
        w0 = g0 / gs
        w1 = g1 / gs

        ti = lax.broadcasted_iota(jnp.int32, (n_tok, n_tok), 0)
        tj = lax.broadcasted_iota(jnp.int32, (n_tok, n_tok), 1)
        low_tri = (ti > tj).astype(jnp.bfloat16)
        ranks = jnp.dot(low_tri, mask_all.astype(jnp.bfloat16),
                        preferred_element_type=jnp.float32).astype(jnp.int32)
        e0r = jnp.transpose(e0c)
        e1r = jnp.transpose(e1c)
        eidT = lax.broadcasted_iota(jnp.int32, (n_exp, n_tok), 0)
        mT = (eidT == e0r) | (eidT == e1r)
        up_tri = (ti < tj).astype(jnp.bfloat16)
        ranksT = jnp.dot(mT.astype(jnp.bfloat16), up_tri,
                         preferred_element_type=jnp.float32).astype(jnp.int32)

        c_row = lax.broadcasted_iota(jnp.int32, (CAP, n_tok), 0)
        s_parts = []
        for e in range(n_exp):
            s_parts.append(
                jnp.where((c_row == ranksT[e:e + 1, :]) & mT[e:e + 1, :],
                          1.0, 0.0).astype(jnp.bfloat16))
        s_all = jnp.concatenate(s_parts, axis=0)
        send_vals = jnp.dot(
            s_all, xv, preferred_element_type=jnp.float32
        ).astype(jnp.bfloat16)
        send_ref[...] = pltpu.roll(
            send_vals,
            lax.rem((N_DEV - my) * e_per * CAP, N_DEV * e_per * CAP),
            axis=0,
        ).reshape(N_DEV, e_per, CAP, d)

        p1 = {}
        for j in range(1, N_DEV):
            dst = lax.rem(my + j, N_DEV)
            dsc = pltpu.make_async_remote_copy(
                src_ref=send_ref.at[pl.ds(j, 1)],
                dst_ref=recv_ref.at[pl.ds(j, 1)],
                send_sem=p1_send.at[j - 1],
                recv_sem=p1_recv.at[j - 1],
                device_id=(dst,),
                device_id_type=pl.DeviceIdType.MESH,
            )
            dsc.start()
            p1[j] = dsc
        recv_ref[pl.ds(0, 1)] = send_ref[pl.ds(0, 1)]

        c_col = lax.broadcasted_iota(jnp.int32, (n_tok, CAP), 1)
        g_parts = []
        for t in range(N_DEV):
            dd = (-t) % N_DEV
            for k in range(e_per):
                e = dd * e_per + k
                ce = (jnp.where(e0c == e, w0, 0.0)
                      + jnp.where(e1c == e, w1, 0.0))
                g_parts.append(
                    jnp.where((c_col == ranks[:, e:e + 1])
                              & mask_all[:, e:e + 1], ce, 0.0
                              ).astype(jnp.bfloat16))
        g_desc = jnp.concatenate(g_parts, axis=1)
        g_m = pltpu.roll(g_desc, my * e_per * CAP, axis=1)
        g_chunks = [
            g_m[:, mc * CH * e_per * CAP:(mc + 1) * CH * e_per * CAP]
            for mc in range(N_CHUNK)
        ]

        wb = ew_ref[...].astype(jnp.bfloat16)
        p2 = {}
        for c in range(N_CHUNK):
            lo = c * CH
            for j in range(max(lo, 1), lo + CH):
                p1[j].wait_recv()
            rv = recv_ref[pl.ds(lo, CH)]
            outs = []
            for k in range(e_per):
                xin = rv[:, k].reshape(CH * CAP, d)
                ok = jnp.dot(xin, wb[k], preferred_element_type=jnp.float32)
                outs.append(ok.astype(jnp.bfloat16).reshape(CH, CAP, h))
            res_ref[pl.ds(lo, CH)] = jnp.stack(outs, axis=1)
            for j in range(max(lo, 1), lo + CH):
                dst = lax.rem(my - j + N_DEV, N_DEV)
                dsc = pltpu.make_async_remote_copy(
                    src_ref=res_ref.at[pl.ds(j, 1)],
                    dst_ref=ret_ref.at[pl.ds(N_DEV - j, 1)],
                    send_sem=p2_send.at[j - 1],
                    recv_sem=p2_recv.at[N_DEV - j - 1],
                    device_id=(dst,),
                    device_id_type=pl.DeviceIdType.MESH,
                )
                dsc.start()
                p2[N_DEV - j] = dsc
        ret_ref[pl.ds(0, 1)] = res_ref[pl.ds(0, 1)]

        acc = jnp.zeros((n_tok, h), jnp.float32)
        for mc in range(N_CHUNK - 1, -1, -1):
            lo = mc * CH
            for m in range(max(lo, 1), lo + CH):
                p2[m].wait_recv()
            rets = ret_ref[pl.ds(lo, CH)].reshape(CH * e_per * CAP, h)
            acc = acc + jnp.dot(g_chunks[mc], rets,
                                preferred_element_type=jnp.float32)
        out_ref[...] = acc

        for j in range(1, N_DEV):
            p1[j].wait_send()
            p2[N_DEV - j].wait_send()

    return pl.pallas_call(
        body,
        out_shape=jax.ShapeDtypeStruct((n_tok, h), jnp.float32),
        in_specs=[
            pl.BlockSpec(memory_space=pltpu.VMEM),
            pl.BlockSpec(memory_space=pltpu.VMEM),
            pl.BlockSpec(memory_space=pltpu.VMEM),
            pl.BlockSpec(memory_space=pltpu.VMEM),
        ],
        out_specs=pl.BlockSpec(memory_space=pltpu.VMEM),
        scratch_shapes=[
            pltpu.VMEM((N_DEV, e_per, CAP, d), jnp.bfloat16),
            pltpu.VMEM((N_DEV, e_per, CAP, d), jnp.bfloat16),
            pltpu.VMEM((N_DEV, e_per, CAP, h), jnp.bfloat16),
            pltpu.VMEM((N_DEV, e_per, CAP, h), jnp.bfloat16),
            pltpu.SemaphoreType.DMA((N_DEV - 1,)),
            pltpu.SemaphoreType.DMA((N_DEV - 1,)),
            pltpu.SemaphoreType.DMA((N_DEV - 1,)),
            pltpu.SemaphoreType.DMA((N_DEV - 1,)),
        ],
        compiler_params=pltpu.CompilerParams(
            collective_id=0,
            vmem_limit_bytes=100 * 1024 * 1024,
        ),
    )(x, router_W, route_idx, expert_W)


# device time: 78521 ns/iter; 1.3848x vs baseline; 1.2750x over previous
import jax
import jax.numpy as jnp
from jax import lax
from jax.experimental import pallas as pl
from jax.experimental.pallas import tpu as pltpu

N_DEV = 32
CAP = 48
N_CHUNK = 4
CH = N_DEV // N_CHUNK


def kernel(x, router_W, route_idx, expert_W):
    n_tok, d = x.shape
    e_per, _, h = expert_W.shape
    n_exp = N_DEV * e_per
    blk = e_per * CAP

    assert e_per == 2

    def body(x_ref, rw_ref, idx_ref, ew_ref, out_ref,
             send_ref, recv_ref, res_ref, ret_ref,
             p1_send, p1_recv, p2_send, p2_recv):
        my = lax.axis_index("i")

        barrier_sem = pltpu.get_barrier_semaphore()
        for off in range(1, N_DEV):
            peer = lax.rem(my + off, N_DEV)
            pl.semaphore_signal(
                barrier_sem, inc=1,
                device_id=(peer,), device_id_type=pl.DeviceIdType.MESH,
            )
        pl.semaphore_wait(barrier_sem, N_DEV - 1)

        xv32 = x_ref[...]
        xv = xv32.astype(jnp.bfloat16)
        scores = jnp.dot(xv32, rw_ref[...], preferred_element_type=jnp.float32)
        mx = jnp.max(scores, axis=-1, keepdims=True)
        p = jnp.exp(scores - mx)
        probs = p / jnp.sum(p, axis=-1, keepdims=True)
        e0c = idx_ref[:, 0:1]
        e1c = idx_ref[:, 1:2]
        eid = lax.broadcasted_iota(jnp.int32, (n_tok, n_exp), 1)
        one0 = eid == e0c
        one1 = eid == e1c
        g0 = jnp.sum(probs * one0.astype(jnp.float32), axis=-1, keepdims=True)
        g1 = jnp.sum(probs * one1.astype(jnp.float32), axis=-1, keepdims=True)
        gs = g0 + g1
        w0 = g0 / gs
        w1 = g1 / gs

        e0r = jnp.transpose(e0c)
        e1r = jnp.transpose(e1c)
        w0r = jnp.transpose(w0)
        w1r = jnp.transpose(w1)
        eidT = lax.broadcasted_iota(jnp.int32, (n_exp, n_tok), 0)
        mT0 = eidT == e0r
        mT1 = eidT == e1r
        mT = mT0 | mT1
        gateT = (jnp.where(mT0, w0r, 0.0)
                 + jnp.where(mT1, w1r, 0.0))
        ti = lax.broadcasted_iota(jnp.int32, (n_tok, n_tok), 0)
        tj = lax.broadcasted_iota(jnp.int32, (n_tok, n_tok), 1)
        up_tri = (ti < tj).astype(jnp.bfloat16)
        ranksT = jnp.dot(mT.astype(jnp.bfloat16), up_tri,
                         preferred_element_type=jnp.float32).astype(jnp.int32)

        c3 = lax.broadcasted_iota(jnp.int32, (n_exp, CAP, n_tok), 1)
        hit3 = (jnp.broadcast_to(ranksT[:, None, :], (n_exp, CAP, n_tok))
                == c3) & jnp.broadcast_to(mT[:, None, :], (n_exp, CAP, n_tok))
        s_all = hit3.astype(jnp.bfloat16).reshape(n_exp * CAP, n_tok)
        gate3 = jnp.broadcast_to(
            gateT[:, None, :], (n_exp, CAP, n_tok))
        gs_all = jnp.where(hit3, gate3, 0.0
                           ).astype(jnp.bfloat16).reshape(n_exp * CAP, n_tok)

        send_ref[...] = jnp.dot(
            s_all, xv, preferred_element_type=jnp.float32
        ).astype(jnp.bfloat16).reshape(N_DEV, e_per, CAP, d)

        for dd in range(N_DEV):
            dsc = pltpu.make_async_remote_copy(
                src_ref=send_ref.at[pl.ds(dd, 1)],
                dst_ref=recv_ref.at[pl.ds(my, 1)],
                send_sem=p1_send.at[dd],
                recv_sem=p1_recv.at[my],
                device_id=(dd,),
                device_id_type=pl.DeviceIdType.MESH,
            )
            pl.when(my != dd)(dsc.start)
        recv_ref[pl.ds(my, 1)] = send_ref[pl.ds(my, 1)]

        def recv_wait(recv_buf, recv_sems, p):
            dsc = pltpu.make_async_remote_copy(
                src_ref=recv_buf.at[pl.ds(0, 1)],
                dst_ref=recv_buf.at[pl.ds(p, 1)],
                send_sem=recv_sems.at[0],
                recv_sem=recv_sems.at[p],
                device_id=(0,),
                device_id_type=pl.DeviceIdType.MESH,
            )
            pl.when(my != p)(dsc.wait_recv)

        wb = ew_ref[...].astype(jnp.bfloat16)
        for c in range(N_CHUNK):
            lo = c * CH
            for src in range(lo, lo + CH):
                recv_wait(recv_ref, p1_recv, src)
            rv = recv_ref[pl.ds(lo, CH)]
            outs = []
            for k in range(e_per):
                xin = rv[:, k].reshape(CH * CAP, d)
                ok = jnp.dot(xin, wb[k], preferred_element_type=jnp.float32)
                outs.append(ok.astype(jnp.bfloat16).reshape(CH, CAP, h))
            res_ref[pl.ds(lo, CH)] = jnp.stack(outs, axis=1)
            for src in range(lo, lo + CH):
                dsc = pltpu.make_async_remote_copy(
                    src_ref=res_ref.at[pl.ds(src, 1)],
                    dst_ref=ret_ref.at[pl.ds(my, 1)],
                    send_sem=p2_send.at[src],
                    recv_sem=p2_recv.at[my],
                    device_id=(src,),
                    device_id_type=pl.DeviceIdType.MESH,
                )
                pl.when(my != src)(dsc.start)
        ret_ref[pl.ds(my, 1)] = res_ref[pl.ds(my, 1)]

        acc = jnp.zeros((n_tok, h), jnp.float32)
        for oc in range(N_CHUNK):
            lo = oc * CH
            for o in range(lo, lo + CH):
                recv_wait(ret_ref, p2_recv, o)
            rets = ret_ref[pl.ds(lo, CH)].reshape(CH * blk, h)
            acc = acc + lax.dot_general(
                gs_all[lo * blk:(lo + CH) * blk, :], rets,
                dimension_numbers=(((0,), (0,)), ((), ())),
                preferred_element_type=jnp.float32,
            )
        out_ref[...] = acc

        for dd in range(N_DEV):
            d1 = pltpu.make_async_remote_copy(
                src_ref=send_ref.at[pl.ds(dd, 1)],
                dst_ref=recv_ref.at[pl.ds(0, 1)],
                send_sem=p1_send.at[dd],
                recv_sem=p1_recv.at[0],
                device_id=(0,),
                device_id_type=pl.DeviceIdType.MESH,
            )
            pl.when(my != dd)(d1.wait_send)
            d2 = pltpu.make_async_remote_copy(
                src_ref=res_ref.at[pl.ds(dd, 1)],
                dst_ref=ret_ref.at[pl.ds(0, 1)],
                send_sem=p2_send.at[dd],
                recv_sem=p2_recv.at[0],
                device_id=(0,),
                device_id_type=pl.DeviceIdType.MESH,
            )
            pl.when(my != dd)(d2.wait_send)

    return pl.pallas_call(
        body,
        out_shape=jax.ShapeDtypeStruct((n_tok, h), jnp.float32),
        in_specs=[
            pl.BlockSpec(memory_space=pltpu.VMEM),
            pl.BlockSpec(memory_space=pltpu.VMEM),
            pl.BlockSpec(memory_space=pltpu.VMEM),
            pl.BlockSpec(memory_space=pltpu.VMEM),
        ],
        out_specs=pl.BlockSpec(memory_space=pltpu.VMEM),
        scratch_shapes=[
            pltpu.VMEM((N_DEV, e_per, CAP, d), jnp.bfloat16),
            pltpu.VMEM((N_DEV, e_per, CAP, d), jnp.bfloat16),
            pltpu.VMEM((N_DEV, e_per, CAP, h), jnp.bfloat16),
            pltpu.VMEM((N_DEV, e_per, CAP, h), jnp.bfloat16),
            pltpu.SemaphoreType.DMA((N_DEV,)),
            pltpu.SemaphoreType.DMA((N_DEV,)),
            pltpu.SemaphoreType.DMA((N_DEV,)),
            pltpu.SemaphoreType.DMA((N_DEV,)),
        ],
        compiler_params=pltpu.CompilerParams(
            collective_id=0,
            vmem_limit_bytes=100 * 1024 * 1024,
        ),
    )(x, router_W, route_idx, expert_W)
